# baseline (speedup 1.0000x reference)
import functools

import jax
import jax.numpy as jnp
from jax import lax
from jax.experimental import pallas as pl
from jax.experimental.pallas import tpu as pltpu
from jax.experimental.pallas import tpu_sc as plsc

try:
    _info = plsc.get_sparse_core_info()
    _NC, _NS, _L = _info.num_cores, _info.num_subcores, _info.num_lanes
except Exception:
    _NC, _NS, _L = 2, 16, 16

_NW = _NC * _NS
_BATCH = 16384
_DIM = 128
_BPW = _BATCH // _NW
_C = 128
_NCHUNK = _BPW // _C


def _make_kernel():
    mesh = plsc.VectorSubcoreMesh(core_axis_name="c", subcore_axis_name="s")

    @functools.partial(
        pl.kernel,
        mesh=mesh,
        compiler_params=pltpu.CompilerParams(needs_layout_passes=False),
        out_type=(
            jax.ShapeDtypeStruct((_BATCH,), jnp.float32),
            jax.ShapeDtypeStruct((_BATCH,), jnp.float32),
        ),
        scratch_types=[
            pltpu.VMEM((_C,), jnp.int32),
            pltpu.VMEM((_C,), jnp.int32),
            pltpu.VMEM((_C,), jnp.int32),
            pltpu.VMEM((_C,), jnp.int32),
            pltpu.VMEM((_C, _DIM), jnp.float32),
            pltpu.VMEM((_C, _DIM), jnp.float32),
            pltpu.VMEM((_C, _DIM), jnp.float32),
            pltpu.VMEM((_C, _DIM), jnp.float32),
            pltpu.VMEM((_BPW,), jnp.float32),
            pltpu.VMEM((_BPW,), jnp.float32),
            pltpu.SemaphoreType.DMA,
        ],
    )
    def transe_kernel(entity_hbm, relation_hbm, heads_hbm, rels_hbm,
                      tails_hbm, negs_hbm, pos_out, neg_out,
                      hidx, ridx, tidx, nidx,
                      hbuf, rbuf, tbuf, nbuf,
                      pos_buf, neg_buf, sem):
        wid = lax.axis_index("s") * _NC + lax.axis_index("c")
        base = wid * _BPW
        lanes = lax.iota(jnp.int32, _L)

        for chunk in range(_NCHUNK):
            cb = chunk * _C
            pltpu.sync_copy(heads_hbm.at[pl.ds(base + cb, _C)], hidx)
            pltpu.sync_copy(rels_hbm.at[pl.ds(base + cb, _C)], ridx)
            pltpu.sync_copy(tails_hbm.at[pl.ds(base + cb, _C)], tidx)
            pltpu.sync_copy(negs_hbm.at[pl.ds(base + cb, _C)], nidx)

            ch = pltpu.async_copy(entity_hbm.at[hidx], hbuf, sem)
            cr = pltpu.async_copy(relation_hbm.at[ridx], rbuf, sem)
            ct = pltpu.async_copy(entity_hbm.at[tidx], tbuf, sem)
            cn = pltpu.async_copy(entity_hbm.at[nidx], nbuf, sem)
            ch.wait()
            cr.wait()
            ct.wait()
            cn.wait()

            for g in range(_C // _L):
                rows = g * _L + lanes

                def body(d, carry):
                    accp, accn = carry
                    col = jnp.full((_L,), d, jnp.int32)
                    hv = plsc.load_gather(hbuf, [rows, col])
                    rv = plsc.load_gather(rbuf, [rows, col])
                    tv = plsc.load_gather(tbuf, [rows, col])
                    nv = plsc.load_gather(nbuf, [rows, col])
                    hr = hv + rv
                    accp = accp + jnp.abs(hr - tv)
                    accn = accn + jnp.abs(hr - nv)
                    return accp, accn

                accp, accn = lax.fori_loop(
                    0, _DIM, body,
                    (jnp.zeros((_L,), jnp.float32),
                     jnp.zeros((_L,), jnp.float32)))
                pos_buf[pl.ds(cb + g * _L, _L)] = accp
                neg_buf[pl.ds(cb + g * _L, _L)] = accn

        pltpu.sync_copy(pos_buf, pos_out.at[pl.ds(base, _BPW)])
        pltpu.sync_copy(neg_buf, neg_out.at[pl.ds(base, _BPW)])

    return transe_kernel


_transe = _make_kernel()


def kernel(entity_emb, relation_emb, heads, relations, tails, negative_tails):
    heads = heads.astype(jnp.int32)
    relations = relations.astype(jnp.int32)
    tails = tails.astype(jnp.int32)
    negative_tails = negative_tails.astype(jnp.int32)
    pos, neg = _transe(entity_emb, relation_emb, heads, relations,
                       tails, negative_tails)
    return (pos, neg)


# speedup vs baseline: 1.1075x; 1.1075x over previous
import functools

import jax
import jax.numpy as jnp
from jax import lax
from jax.experimental import pallas as pl
from jax.experimental.pallas import tpu as pltpu
from jax.experimental.pallas import tpu_sc as plsc

try:
    _info = plsc.get_sparse_core_info()
    _NC, _NS, _L = _info.num_cores, _info.num_subcores, _info.num_lanes
except Exception:
    _NC, _NS, _L = 2, 16, 16

_NW = _NC * _NS
_BATCH = 16384
_DIM = 128
_BPW = _BATCH // _NW
_C = 64
_NCHUNK = _BPW // _C
_G = _C // _L
_U = 4


def _make_kernel():
    mesh = plsc.VectorSubcoreMesh(core_axis_name="c", subcore_axis_name="s")

    @functools.partial(
        pl.kernel,
        mesh=mesh,
        compiler_params=pltpu.CompilerParams(needs_layout_passes=False),
        out_type=(
            jax.ShapeDtypeStruct((_BATCH,), jnp.float32),
            jax.ShapeDtypeStruct((_BATCH,), jnp.float32),
        ),
        scratch_types=[
            pltpu.VMEM((_BPW,), jnp.int32),
            pltpu.VMEM((_BPW,), jnp.int32),
            pltpu.VMEM((_BPW,), jnp.int32),
            pltpu.VMEM((_BPW,), jnp.int32),
            pltpu.VMEM((_C, _DIM), jnp.float32),
            pltpu.VMEM((_C, _DIM), jnp.float32),
            pltpu.VMEM((_C, _DIM), jnp.float32),
            pltpu.VMEM((_C, _DIM), jnp.float32),
            pltpu.VMEM((_C, _DIM), jnp.float32),
            pltpu.VMEM((_C, _DIM), jnp.float32),
            pltpu.VMEM((_C, _DIM), jnp.float32),
            pltpu.VMEM((_C, _DIM), jnp.float32),
            pltpu.VMEM((_BPW,), jnp.float32),
            pltpu.VMEM((_BPW,), jnp.float32),
            pltpu.SemaphoreType.DMA,
            pltpu.SemaphoreType.DMA,
        ],
    )
    def transe_kernel(entity_hbm, relation_hbm, heads_hbm, rels_hbm,
                      tails_hbm, negs_hbm, pos_out, neg_out,
                      hidx, ridx, tidx, nidx,
                      hb0, rb0, tb0, nb0, hb1, rb1, tb1, nb1,
                      pos_buf, neg_buf, sem0, sem1):
        wid = lax.axis_index("s") * _NC + lax.axis_index("c")
        base = wid * _BPW
        lanes = lax.iota(jnp.int32, _L)

        pltpu.sync_copy(heads_hbm.at[pl.ds(base, _BPW)], hidx)
        pltpu.sync_copy(rels_hbm.at[pl.ds(base, _BPW)], ridx)
        pltpu.sync_copy(tails_hbm.at[pl.ds(base, _BPW)], tidx)
        pltpu.sync_copy(negs_hbm.at[pl.ds(base, _BPW)], nidx)

        bufs = ((hb0, rb0, tb0, nb0, sem0), (hb1, rb1, tb1, nb1, sem1))

        def issue(c):
            hb, rb, tb, nb, sem = bufs[c % 2]
            cb = c * _C
            return (
                pltpu.async_copy(entity_hbm.at[hidx.at[pl.ds(cb, _C)]],
                                 hb, sem),
                pltpu.async_copy(relation_hbm.at[ridx.at[pl.ds(cb, _C)]],
                                 rb, sem),
                pltpu.async_copy(entity_hbm.at[tidx.at[pl.ds(cb, _C)]],
                                 tb, sem),
                pltpu.async_copy(entity_hbm.at[nidx.at[pl.ds(cb, _C)]],
                                 nb, sem),
            )

        rows = [g * _L + lanes for g in range(_G)]
        zero = jnp.zeros((_L,), jnp.float32)

        pending = issue(0)
        for c in range(_NCHUNK):
            nxt = issue(c + 1) if c + 1 < _NCHUNK else None
            for cp in pending:
                cp.wait()
            hb, rb, tb, nb, _ = bufs[c % 2]

            def body(i, accs):
                accs = list(accs)
                d0 = i * _U
                for u in range(_U):
                    col = jnp.full((_L,), d0 + u, jnp.int32)
                    for g in range(_G):
                        hv = plsc.load_gather(hb, [rows[g], col])
                        rv = plsc.load_gather(rb, [rows[g], col])
                        tv = plsc.load_gather(tb, [rows[g], col])
                        nv = plsc.load_gather(nb, [rows[g], col])
                        hr = hv + rv
                        accs[2 * g] = accs[2 * g] + jnp.abs(hr - tv)
                        accs[2 * g + 1] = accs[2 * g + 1] + jnp.abs(hr - nv)
                return tuple(accs)

            accs = lax.fori_loop(0, _DIM // _U, body, (zero,) * (2 * _G))
            cb = c * _C
            for g in range(_G):
                pos_buf[pl.ds(cb + g * _L, _L)] = accs[2 * g]
                neg_buf[pl.ds(cb + g * _L, _L)] = accs[2 * g + 1]
            pending = nxt

        pltpu.sync_copy(pos_buf, pos_out.at[pl.ds(base, _BPW)])
        pltpu.sync_copy(neg_buf, neg_out.at[pl.ds(base, _BPW)])

    return transe_kernel


_transe = _make_kernel()


def kernel(entity_emb, relation_emb, heads, relations, tails, negative_tails):
    heads = heads.astype(jnp.int32)
    relations = relations.astype(jnp.int32)
    tails = tails.astype(jnp.int32)
    negative_tails = negative_tails.astype(jnp.int32)
    pos, neg = _transe(entity_emb, relation_emb, heads, relations,
                       tails, negative_tails)
    return (pos, neg)


# speedup vs baseline: 4.0802x; 3.6840x over previous
import functools

import jax
import jax.numpy as jnp
from jax import lax
from jax.experimental import pallas as pl
from jax.experimental.pallas import tpu as pltpu
from jax.experimental.pallas import tpu_sc as plsc

try:
    _info = plsc.get_sparse_core_info()
    _NC, _NS, _L = _info.num_cores, _info.num_subcores, _info.num_lanes
except Exception:
    _NC, _NS, _L = 2, 16, 16

_NW = _NC * _NS
_BATCH = 16384
_DIM = 128
_BPW = _BATCH // _NW
_C = 64
_NCHUNK = _BPW // _C
_G = _C // _L
_U = 4


def _make_kernel():
    mesh = plsc.VectorSubcoreMesh(core_axis_name="c", subcore_axis_name="s")

    @functools.partial(
        pl.kernel,
        mesh=mesh,
        compiler_params=pltpu.CompilerParams(needs_layout_passes=False),
        out_type=(
            jax.ShapeDtypeStruct((_BATCH,), jnp.float32),
            jax.ShapeDtypeStruct((_BATCH,), jnp.float32),
        ),
        scratch_types=[
            pltpu.VMEM((_BPW,), jnp.int32),
            pltpu.VMEM((_BPW,), jnp.int32),
            pltpu.VMEM((_BPW,), jnp.int32),
            pltpu.VMEM((_BPW,), jnp.int32),
            pltpu.VMEM((_C, _DIM), jnp.float32),
            pltpu.VMEM((_C, _DIM), jnp.float32),
            pltpu.VMEM((_C, _DIM), jnp.float32),
            pltpu.VMEM((_C, _DIM), jnp.float32),
            pltpu.VMEM((_C, _DIM), jnp.float32),
            pltpu.VMEM((_C, _DIM), jnp.float32),
            pltpu.VMEM((_C, _DIM), jnp.float32),
            pltpu.VMEM((_C, _DIM), jnp.float32),
            pltpu.VMEM((_BPW,), jnp.float32),
            pltpu.VMEM((_BPW,), jnp.float32),
            pltpu.SemaphoreType.DMA,
            pltpu.SemaphoreType.DMA,
        ],
    )
    def transe_kernel(entity_hbm, relation_hbm, heads_hbm, rels_hbm,
                      tails_hbm, negs_hbm, pos_out, neg_out,
                      hidx, ridx, tidx, nidx,
                      hb0, rb0, tb0, nb0, hb1, rb1, tb1, nb1,
                      pos_buf, neg_buf, sem0, sem1):
        wid = lax.axis_index("s") * _NC + lax.axis_index("c")
        base = wid * _BPW
        lanes = lax.iota(jnp.int32, _L)

        pltpu.sync_copy(heads_hbm.at[pl.ds(base, _BPW)], hidx)
        pltpu.sync_copy(rels_hbm.at[pl.ds(base, _BPW)], ridx)
        pltpu.sync_copy(tails_hbm.at[pl.ds(base, _BPW)], tidx)
        pltpu.sync_copy(negs_hbm.at[pl.ds(base, _BPW)], nidx)

        bufs = ((hb0, rb0, tb0, nb0, sem0), (hb1, rb1, tb1, nb1, sem1))

        def issue(c):
            hb, rb, tb, nb, sem = bufs[c % 2]
            cb = c * _C
            return (
                pltpu.async_copy(entity_hbm.at[hidx.at[pl.ds(cb, _C)]],
                                 hb, sem),
                pltpu.async_copy(relation_hbm.at[ridx.at[pl.ds(cb, _C)]],
                                 rb, sem),
                pltpu.async_copy(entity_hbm.at[tidx.at[pl.ds(cb, _C)]],
                                 tb, sem),
                pltpu.async_copy(entity_hbm.at[nidx.at[pl.ds(cb, _C)]],
                                 nb, sem),
            )

        last = lanes == (_L - 1)

        pending = issue(0)
        for c in range(_NCHUNK):
            nxt = issue(c + 1) if c + 1 < _NCHUNK else None
            for cp in pending:
                cp.wait()
            hb, rb, tb, nb, _ = bufs[c % 2]
            cb = c * _C

            def body(i, carry):
                accp = jnp.zeros((_L,), jnp.float32)
                accn = jnp.zeros((_L,), jnp.float32)
                for j in range(_DIM // _L):
                    sl = pl.ds(j * _L, _L)
                    hr = hb[i, sl] + rb[i, sl]
                    accp = accp + jnp.abs(hr - tb[i, sl])
                    accn = accn + jnp.abs(hr - nb[i, sl])
                out_idx = jnp.full((_L,), cb + i, jnp.int32)
                plsc.store_scatter(pos_buf, [out_idx], plsc.cumsum(accp),
                                   mask=last)
                plsc.store_scatter(neg_buf, [out_idx], plsc.cumsum(accn),
                                   mask=last)
                return carry

            lax.fori_loop(0, _C, body, 0)
            pending = nxt

        pltpu.sync_copy(pos_buf, pos_out.at[pl.ds(base, _BPW)])
        pltpu.sync_copy(neg_buf, neg_out.at[pl.ds(base, _BPW)])

    return transe_kernel


_transe = _make_kernel()


def kernel(entity_emb, relation_emb, heads, relations, tails, negative_tails):
    heads = heads.astype(jnp.int32)
    relations = relations.astype(jnp.int32)
    tails = tails.astype(jnp.int32)
    negative_tails = negative_tails.astype(jnp.int32)
    pos, neg = _transe(entity_emb, relation_emb, heads, relations,
                       tails, negative_tails)
    return (pos, neg)
